# Initial kernel scaffold; baseline (speedup 1.0000x reference)
import functools

import jax
import jax.numpy as jnp
from jax import lax
from jax.experimental import pallas as pl
from jax.experimental.pallas import tpu as pltpu
from jax.experimental.pallas import tpu_sc as plsc

_NUM_WORKERS = 32
_CHUNK = 128


@functools.cache
def _make_gather(total: int, embed: int):
    assert total % (_NUM_WORKERS * _CHUNK) == 0
    chunks_per_worker = total // (_NUM_WORKERS * _CHUNK)
    bpw = chunks_per_worker * _CHUNK

    mesh = plsc.VectorSubcoreMesh(core_axis_name="c", subcore_axis_name="s")

    @functools.partial(
        pl.kernel,
        mesh=mesh,
        out_type=jax.ShapeDtypeStruct((total, embed), jnp.float32),
        scratch_types=[
            pltpu.VMEM((chunks_per_worker, _CHUNK), jnp.int32),
            pltpu.VMEM((_CHUNK, embed), jnp.float32),
            pltpu.SemaphoreType.DMA,
        ],
    )
    def gather_kernel(idx_hbm, table_hbm, out_hbm, idx_v, rows, sem):
        wid = lax.axis_index("s") * 2 + lax.axis_index("c")
        base = wid * bpw
        pltpu.sync_copy(idx_hbm.at[wid], idx_v)

        def loop_body(j, carry):
            pltpu.async_copy(table_hbm.at[idx_v.at[j]], rows, sem).wait()
            pltpu.sync_copy(rows, out_hbm.at[pl.ds(base + j * _CHUNK, _CHUNK)])
            return carry

        lax.fori_loop(0, chunks_per_worker, loop_body, 0)

    return gather_kernel


def kernel(cleavage_indices, pos_embed):
    b, s = cleavage_indices.shape
    embed = pos_embed.shape[1]
    total = b * s
    chunks_per_worker = total // (_NUM_WORKERS * _CHUNK)
    idx = cleavage_indices.astype(jnp.int32).reshape(
        _NUM_WORKERS, chunks_per_worker * _CHUNK
    )
    out = _make_gather(total, embed)(idx, pos_embed)
    return out.reshape(b, s, embed)


# speedup vs baseline: 2.9145x; 2.9145x over previous
import functools

import jax
import jax.numpy as jnp
from jax import lax
from jax.experimental import pallas as pl
from jax.experimental.pallas import tpu as pltpu
from jax.experimental.pallas import tpu_sc as plsc

_NUM_WORKERS = 32
_CHUNK = 128


@functools.cache
def _make_gather(total: int, embed: int):
    assert total % (_NUM_WORKERS * _CHUNK) == 0
    chunks_per_worker = total // (_NUM_WORKERS * _CHUNK)
    bpw = chunks_per_worker * _CHUNK

    mesh = plsc.VectorSubcoreMesh(core_axis_name="c", subcore_axis_name="s")

    @functools.partial(
        pl.kernel,
        mesh=mesh,
        out_type=jax.ShapeDtypeStruct((total, embed), jnp.float32),
        scratch_types=[
            pltpu.VMEM((chunks_per_worker, _CHUNK), jnp.int32),
            pltpu.VMEM((_CHUNK, embed), jnp.float32),
            pltpu.SemaphoreType.DMA,
        ],
    )
    def gather_kernel(idx_hbm, table_hbm, out_hbm, idx_v, rows, sem):
        wid = lax.axis_index("s") * 2 + lax.axis_index("c")
        base = wid * bpw
        pltpu.sync_copy(idx_hbm.at[wid], idx_v)

        def loop_body(j, carry):
            pltpu.async_copy(table_hbm.at[idx_v.at[j]], rows, sem).wait()
            pltpu.sync_copy(rows, out_hbm.at[pl.ds(base + j * _CHUNK, _CHUNK)])
            return carry

        lax.fori_loop(0, chunks_per_worker, loop_body, 0)

    return gather_kernel


def kernel(cleavage_indices, pos_embed):
    b, s = cleavage_indices.shape
    embed = pos_embed.shape[1]
    total = b * s
    chunks_per_worker = total // (_NUM_WORKERS * _CHUNK)
    idx = cleavage_indices.astype(jnp.int32).reshape(
        _NUM_WORKERS, chunks_per_worker, _CHUNK
    )
    out = _make_gather(total, embed)(idx, pos_embed)
    return out.reshape(b, s, embed)


# speedup vs baseline: 3.2538x; 1.1164x over previous
import functools

import jax
import jax.numpy as jnp
from jax import lax
from jax.experimental import pallas as pl
from jax.experimental.pallas import tpu as pltpu
from jax.experimental.pallas import tpu_sc as plsc

_NUM_WORKERS = 32
_CHUNK = 128
_NBUF = 5
_LOOKAHEAD = 3


@functools.cache
def _make_gather(total: int, embed: int):
    assert total % (_NUM_WORKERS * _CHUNK) == 0
    nchunks = total // (_NUM_WORKERS * _CHUNK)
    assert nchunks % _NBUF == 0 and _LOOKAHEAD < _NBUF
    bpw = nchunks * _CHUNK

    mesh = plsc.VectorSubcoreMesh(core_axis_name="c", subcore_axis_name="s")

    row_buf = jax.ShapeDtypeStruct((_CHUNK, embed), jnp.float32)

    @functools.partial(
        pl.kernel,
        mesh=mesh,
        out_type=jax.ShapeDtypeStruct((total, embed), jnp.float32),
        scratch_types=[
            pltpu.VMEM((nchunks, _CHUNK), jnp.int32),
        ]
        + [pltpu.VMEM(row_buf.shape, row_buf.dtype) for _ in range(_NBUF)]
        + [pltpu.SemaphoreType.DMA for _ in range(2 * _NBUF)],
    )
    def gather_kernel(idx_hbm, table_hbm, out_hbm, idx_v, *bufs_and_sems):
        rows = bufs_and_sems[:_NBUF]
        gsem = bufs_and_sems[_NBUF : 2 * _NBUF]
        wsem = bufs_and_sems[2 * _NBUF :]

        wid = lax.axis_index("s") * 2 + lax.axis_index("c")
        base = wid * bpw
        pltpu.sync_copy(idx_hbm.at[wid], idx_v)

        def gather_copy(j, slot):
            return pltpu.make_async_copy(
                table_hbm.at[idx_v.at[j]], rows[slot], gsem[slot]
            )

        def wb_copy(j, slot):
            return pltpu.make_async_copy(
                rows[slot], out_hbm.at[pl.ds(base + j * _CHUNK, _CHUNK)], wsem[slot]
            )

        for j in range(_LOOKAHEAD):
            gather_copy(j, j).start()

        def group(i, carry):
            for b in range(_NBUF):
                j = i * _NBUF + b
                gather_copy(j, b).wait()
                wb_copy(j, b).start()
                s2 = (b + _LOOKAHEAD) % _NBUF
                jn = j + _LOOKAHEAD

                @pl.when((jn >= _NBUF) & (jn < nchunks))
                def _():
                    wb_copy(jn - _NBUF, s2).wait()

                @pl.when(jn < nchunks)
                def _():
                    gather_copy(jn, s2).start()

            return carry

        lax.fori_loop(0, nchunks // _NBUF, group, 0)

        for b in range(_NBUF):
            wb_copy(nchunks - _NBUF + b, b).wait()

    return gather_kernel


def kernel(cleavage_indices, pos_embed):
    b, s = cleavage_indices.shape
    embed = pos_embed.shape[1]
    total = b * s
    nchunks = total // (_NUM_WORKERS * _CHUNK)
    idx = cleavage_indices.astype(jnp.int32).reshape(_NUM_WORKERS, nchunks, _CHUNK)
    out = _make_gather(total, embed)(idx, pos_embed)
    return out.reshape(b, s, embed)


# speedup vs baseline: 5.7407x; 1.7643x over previous
import functools

import jax
import jax.numpy as jnp
from jax import lax
from jax.experimental import pallas as pl
from jax.experimental.pallas import tpu as pltpu
from jax.experimental.pallas import tpu_sc as plsc

_NUM_WORKERS = 32
_NBUF = 8
_LOOKAHEAD = 4


@functools.cache
def _make_gather(batch: int, seq: int, embed: int):
    assert batch % (_NUM_WORKERS * _NBUF) == 0 and _LOOKAHEAD < _NBUF
    bpw = batch // _NUM_WORKERS

    mesh = plsc.VectorSubcoreMesh(core_axis_name="c", subcore_axis_name="s")

    @functools.partial(
        pl.kernel,
        mesh=mesh,
        out_type=jax.ShapeDtypeStruct((batch, seq, embed), jnp.float32),
        scratch_types=[
            pltpu.VMEM((bpw, seq), jnp.int32),
        ]
        + [pltpu.VMEM((seq, embed), jnp.float32) for _ in range(_NBUF)]
        + [pltpu.SemaphoreType.DMA for _ in range(2 * _NBUF)],
    )
    def gather_kernel(idx_hbm, table_hbm, out_hbm, idx_v, *bufs_and_sems):
        rows = bufs_and_sems[:_NBUF]
        gsem = bufs_and_sems[_NBUF : 2 * _NBUF]
        wsem = bufs_and_sems[2 * _NBUF :]

        wid = lax.axis_index("s") * 2 + lax.axis_index("c")
        base = wid * bpw
        pltpu.sync_copy(idx_hbm.at[wid], idx_v)

        def gather_copy(j, slot):
            return pltpu.make_async_copy(
                table_hbm.at[idx_v.at[j]], rows[slot], gsem[slot]
            )

        def wb_copy(j, slot):
            return pltpu.make_async_copy(rows[slot], out_hbm.at[base + j], wsem[slot])

        for j in range(_LOOKAHEAD):
            gather_copy(j, j).start()

        def group(i, carry):
            for b in range(_NBUF):
                j = i * _NBUF + b
                gather_copy(j, b).wait()
                wb_copy(j, b).start()
                s2 = (b + _LOOKAHEAD) % _NBUF
                jn = j + _LOOKAHEAD

                @pl.when((jn >= _NBUF) & (jn < bpw))
                def _():
                    wb_copy(jn - _NBUF, s2).wait()

                @pl.when(jn < bpw)
                def _():
                    gather_copy(jn, s2).start()

            return carry

        lax.fori_loop(0, bpw // _NBUF, group, 0)

        for b in range(_NBUF):
            wb_copy(bpw - _NBUF + b, b).wait()

    return gather_kernel


def kernel(cleavage_indices, pos_embed):
    b, s = cleavage_indices.shape
    embed = pos_embed.shape[1]
    idx = cleavage_indices.astype(jnp.int32).reshape(
        _NUM_WORKERS, b // _NUM_WORKERS, s
    )
    return _make_gather(b, s, embed)(idx, pos_embed)


# speedup vs baseline: 5.7476x; 1.0012x over previous
import functools

import jax
import jax.numpy as jnp
from jax import lax
from jax.experimental import pallas as pl
from jax.experimental.pallas import tpu as pltpu
from jax.experimental.pallas import tpu_sc as plsc

_NUM_WORKERS = 32
_NBUF = 8
_LOOKAHEAD = 4


@functools.cache
def _make_gather(batch: int, seq: int, embed: int):
    assert batch % (_NUM_WORKERS * _NBUF) == 0 and _LOOKAHEAD < _NBUF
    bpw = batch // _NUM_WORKERS

    mesh = plsc.VectorSubcoreMesh(core_axis_name="c", subcore_axis_name="s")

    @functools.partial(
        pl.kernel,
        mesh=mesh,
        out_type=jax.ShapeDtypeStruct((batch, seq, embed), jnp.float32),
        scratch_types=[
            pltpu.VMEM((bpw, seq), jnp.int32),
        ]
        + [pltpu.VMEM((seq, embed), jnp.float32) for _ in range(_NBUF)]
        + [pltpu.SemaphoreType.DMA for _ in range(2 * _NBUF)],
        compiler_params=pltpu.CompilerParams(use_tc_tiling_on_sc=True),
    )
    def gather_kernel(idx_hbm, table_hbm, out_hbm, idx_v, *bufs_and_sems):
        rows = bufs_and_sems[:_NBUF]
        gsem = bufs_and_sems[_NBUF : 2 * _NBUF]
        wsem = bufs_and_sems[2 * _NBUF :]

        wid = lax.axis_index("s") * 2 + lax.axis_index("c")
        base = wid * bpw
        pltpu.sync_copy(idx_hbm.at[wid], idx_v)

        def gather_copy(j, slot):
            return pltpu.make_async_copy(
                table_hbm.at[idx_v.at[j]], rows[slot], gsem[slot]
            )

        def wb_copy(j, slot):
            return pltpu.make_async_copy(rows[slot], out_hbm.at[base + j], wsem[slot])

        for j in range(_LOOKAHEAD):
            gather_copy(j, j).start()

        def group(i, carry):
            for b in range(_NBUF):
                j = i * _NBUF + b
                gather_copy(j, b).wait()
                wb_copy(j, b).start()
                s2 = (b + _LOOKAHEAD) % _NBUF
                jn = j + _LOOKAHEAD

                @pl.when((jn >= _NBUF) & (jn < bpw))
                def _():
                    wb_copy(jn - _NBUF, s2).wait()

                @pl.when(jn < bpw)
                def _():
                    gather_copy(jn, s2).start()

            return carry

        lax.fori_loop(0, bpw // _NBUF, group, 0)

        for b in range(_NBUF):
            wb_copy(bpw - _NBUF + b, b).wait()

    return gather_kernel


def kernel(cleavage_indices, pos_embed):
    b, s = cleavage_indices.shape
    embed = pos_embed.shape[1]
    idx = cleavage_indices.astype(jnp.int32).reshape(
        _NUM_WORKERS, b // _NUM_WORKERS, s
    )
    return _make_gather(b, s, embed)(idx, pos_embed)


# speedup vs baseline: 9.7861x; 1.7026x over previous
import functools

import jax
import jax.numpy as jnp
from jax import lax
from jax.experimental import pallas as pl
from jax.experimental.pallas import tpu as pltpu
from jax.experimental.pallas import tpu_sc as plsc

_NUM_WORKERS = 32
_NBUF = 5
_LOOKAHEAD = 3


@functools.cache
def _make_gather(batch: int, seq: int, embed: int):
    assert batch % _NUM_WORKERS == 0 and _LOOKAHEAD < _NBUF
    assert seq % _NBUF == 0
    bpw = batch // _NUM_WORKERS

    mesh = plsc.VectorSubcoreMesh(core_axis_name="c", subcore_axis_name="s")

    @functools.partial(
        pl.kernel,
        mesh=mesh,
        out_type=jax.ShapeDtypeStruct((seq, batch, embed), jnp.float32),
        scratch_types=[
            pltpu.VMEM((seq, bpw), jnp.int32),
        ]
        + [pltpu.VMEM((bpw, embed), jnp.float32) for _ in range(_NBUF)]
        + [pltpu.SemaphoreType.DMA for _ in range(2 * _NBUF)],
    )
    def gather_kernel(idx_hbm, table_hbm, out_hbm, idx_v, *bufs_and_sems):
        rows = bufs_and_sems[:_NBUF]
        gsem = bufs_and_sems[_NBUF : 2 * _NBUF]
        wsem = bufs_and_sems[2 * _NBUF :]

        wid = lax.axis_index("s") * 2 + lax.axis_index("c")
        base = wid * bpw
        pltpu.sync_copy(idx_hbm.at[wid], idx_v)

        def gather_copy(j, slot):
            return pltpu.make_async_copy(
                table_hbm.at[idx_v.at[j]], rows[slot], gsem[slot]
            )

        def wb_copy(j, slot):
            return pltpu.make_async_copy(
                rows[slot], out_hbm.at[j, pl.ds(base, bpw)], wsem[slot]
            )

        for j in range(_LOOKAHEAD):
            gather_copy(j, j).start()

        def group(i, carry):
            for b in range(_NBUF):
                j = i * _NBUF + b
                gather_copy(j, b).wait()
                wb_copy(j, b).start()
                s2 = (b + _LOOKAHEAD) % _NBUF
                jn = j + _LOOKAHEAD

                @pl.when((jn >= _NBUF) & (jn < seq))
                def _():
                    wb_copy(jn - _NBUF, s2).wait()

                @pl.when(jn < seq)
                def _():
                    gather_copy(jn, s2).start()

            return carry

        lax.fori_loop(0, seq // _NBUF, group, 0)

        for b in range(_NBUF):
            wb_copy(seq - _NBUF + b, b).wait()

    return gather_kernel


def kernel(cleavage_indices, pos_embed):
    b, s = cleavage_indices.shape
    embed = pos_embed.shape[1]
    bpw = b // _NUM_WORKERS
    idx3 = (
        cleavage_indices.astype(jnp.int32)
        .reshape(_NUM_WORKERS, bpw, s)
        .transpose(0, 2, 1)
    )
    out = _make_gather(b, s, embed)(idx3, pos_embed)
    return out.transpose(1, 0, 2)
